# Initial kernel scaffold; baseline (speedup 1.0000x reference)
import jax
import jax.numpy as jnp
from jax.experimental import pallas as pl


def kernel(x, Wr, W1, b1, W2, b2):
    raise NotImplementedError("write your pallas kernel here")


# speedup vs baseline: 2.5433x; 2.5433x over previous
import functools

import jax
import jax.numpy as jnp
from jax import lax
from jax.experimental import pallas as pl
from jax.experimental.pallas import tpu as pltpu
from jax.experimental.pallas import tpu_sc as plsc

D = 1024
NE = 64
NTOK = 2048
TM = 64
NPAD = 8192
NT = NPAD // TM
NW = 32
GCH = 64
CCH = 32


def _router_body(x_ref, wr_ref, i1_ref, i2_ref, w1_ref, w2_ref):
    xb = x_ref[...]
    wr = wr_ref[...]
    logits = lax.dot_general(xb, wr, (((1,), (1,)), ((), ())),
                             preferred_element_type=jnp.float32)
    iota = lax.broadcasted_iota(jnp.int32, logits.shape, 1)
    m1 = jnp.max(logits, axis=1, keepdims=True)
    i1 = jnp.min(jnp.where(logits == m1, iota, NE), axis=1, keepdims=True)
    masked = jnp.where(iota == i1, -jnp.inf, logits)
    m2 = jnp.max(masked, axis=1, keepdims=True)
    i2 = jnp.min(jnp.where(masked == m2, iota, NE), axis=1, keepdims=True)
    w1 = 1.0 / (1.0 + jnp.exp(m2 - m1))
    i1_ref[...] = i1
    i2_ref[...] = i2
    w1_ref[...] = w1
    w2_ref[...] = 1.0 - w1


def _router(x2, Wr):
    return pl.pallas_call(
        _router_body,
        out_shape=[
            jax.ShapeDtypeStruct((NTOK, 1), jnp.int32),
            jax.ShapeDtypeStruct((NTOK, 1), jnp.int32),
            jax.ShapeDtypeStruct((NTOK, 1), jnp.float32),
            jax.ShapeDtypeStruct((NTOK, 1), jnp.float32),
        ],
    )(x2, Wr)


def _gather_body(x_hbm, rt_hbm, out_hbm, idx_v, rows_v, sem):
    wid = lax.axis_index("s") * 2 + lax.axis_index("c")
    base = wid * (NPAD // NW)

    def chunk(c, carry):
        off = base + c * GCH
        pltpu.sync_copy(rt_hbm.at[pl.ds(off, GCH)], idx_v)
        pltpu.async_copy(x_hbm.at[idx_v], rows_v, sem).wait()
        pltpu.sync_copy(rows_v, out_hbm.at[pl.ds(off, GCH)])
        return carry

    lax.fori_loop(0, (NPAD // NW) // GCH, chunk, 0)


def _gather(x2, row_token):
    f = functools.partial(
        pl.kernel,
        mesh=plsc.VectorSubcoreMesh(core_axis_name="c", subcore_axis_name="s"),
        out_type=jax.ShapeDtypeStruct((NPAD, D), jnp.float32),
        scratch_types=[
            pltpu.VMEM((GCH,), jnp.int32),
            pltpu.VMEM((GCH, D), jnp.float32),
            pltpu.SemaphoreType.DMA,
        ],
    )(_gather_body)
    return f(x2, row_token)


_RSQRT2 = 0.7071067811865476


def _ffn_body(te_ref, nt_ref, xs_ref, w1_ref, b1_ref, w2_ref, b2_ref, rw_ref,
              ys_ref):
    j = pl.program_id(0)

    @pl.when(j < nt_ref[0])
    def _():
        xb = xs_ref[...]
        h = lax.dot_general(xb, w1_ref[0], (((1,), (1,)), ((), ())),
                            preferred_element_type=jnp.float32)
        h = h + b1_ref[0]
        h = 0.5 * h * (1.0 + lax.erf(h * _RSQRT2))
        y = lax.dot_general(h, w2_ref[0], (((1,), (1,)), ((), ())),
                            preferred_element_type=jnp.float32)
        y = y + b2_ref[0]
        ys_ref[...] = y * rw_ref[0, 0, :][:, None]


def _ffn(tile_expert, nactive, xs, W1, b1, W2, b2, rw3):
    def _jm(j, te, nt):
        return jnp.minimum(j, nt[0] - 1)

    grid_spec = pltpu.PrefetchScalarGridSpec(
        num_scalar_prefetch=2,
        grid=(NT,),
        in_specs=[
            pl.BlockSpec((TM, D), lambda j, te, nt: (_jm(j, te, nt), 0)),
            pl.BlockSpec((1, D, D), lambda j, te, nt: (te[_jm(j, te, nt)], 0, 0)),
            pl.BlockSpec((1, 1, D), lambda j, te, nt: (te[_jm(j, te, nt)], 0, 0)),
            pl.BlockSpec((1, D, D), lambda j, te, nt: (te[_jm(j, te, nt)], 0, 0)),
            pl.BlockSpec((1, 1, D), lambda j, te, nt: (te[_jm(j, te, nt)], 0, 0)),
            pl.BlockSpec((1, 1, TM), lambda j, te, nt: (_jm(j, te, nt), 0, 0)),
        ],
        out_specs=pl.BlockSpec((TM, D), lambda j, te, nt: (_jm(j, te, nt), 0)),
    )
    return pl.pallas_call(
        _ffn_body,
        grid_spec=grid_spec,
        out_shape=jax.ShapeDtypeStruct((NPAD, D), jnp.float32),
        compiler_params=pltpu.CompilerParams(
            dimension_semantics=("arbitrary",)),
    )(tile_expert, nactive, xs, W1, b1.reshape(NE, 1, D), W2,
      b2.reshape(NE, 1, D), rw3)


def _combine_body(ys_hbm, pa_hbm, pb_hbm, out_hbm, ia_v, ib_v, ba_v, bb_v,
                  sa, sb):
    wid = lax.axis_index("s") * 2 + lax.axis_index("c")
    base = wid * (NTOK // NW)

    def chunk(c, carry):
        off = base + c * CCH
        pltpu.sync_copy(pa_hbm.at[pl.ds(off, CCH)], ia_v)
        pltpu.sync_copy(pb_hbm.at[pl.ds(off, CCH)], ib_v)
        cpa = pltpu.async_copy(ys_hbm.at[ia_v], ba_v, sa)
        cpb = pltpu.async_copy(ys_hbm.at[ib_v], bb_v, sb)
        cpa.wait()
        cpb.wait()

        def row(r, carry2):
            for i in range(D // 16):
                sl = pl.ds(i * 16, 16)
                ba_v[r, sl] = ba_v[r, sl] + bb_v[r, sl]
            return carry2

        lax.fori_loop(0, CCH, row, 0)
        pltpu.sync_copy(ba_v, out_hbm.at[pl.ds(off, CCH)])
        return carry

    lax.fori_loop(0, (NTOK // NW) // CCH, chunk, 0)


def _combine(ys, posA, posB):
    f = functools.partial(
        pl.kernel,
        mesh=plsc.VectorSubcoreMesh(core_axis_name="c", subcore_axis_name="s"),
        out_type=jax.ShapeDtypeStruct((NTOK, D), jnp.float32),
        scratch_types=[
            pltpu.VMEM((CCH,), jnp.int32),
            pltpu.VMEM((CCH,), jnp.int32),
            pltpu.VMEM((CCH, D), jnp.float32),
            pltpu.VMEM((CCH, D), jnp.float32),
            pltpu.SemaphoreType.DMA,
            pltpu.SemaphoreType.DMA,
        ],
    )(_combine_body)
    return f(ys, posA, posB)


def _dispatch_plan(i1, i2, w1, w2):
    e = jnp.concatenate([i1, i2])
    oh = (e[:, None] == jnp.arange(NE, dtype=jnp.int32)[None, :])
    cum = jnp.cumsum(oh.astype(jnp.int32), axis=0)
    counts = cum[-1]
    rank = jnp.take_along_axis(cum, e[:, None], axis=1)[:, 0] - 1
    padded = ((counts + TM - 1) // TM) * TM
    pad_end = jnp.cumsum(padded)
    pad_off = pad_end - padded
    pos = pad_off[e] + rank

    token_ids = jnp.tile(jnp.arange(NTOK, dtype=jnp.int32), 2)
    row_token = jnp.zeros((NPAD,), jnp.int32).at[pos].set(token_ids)
    w_all = jnp.concatenate([w1, w2])
    row_w = jnp.zeros((NPAD,), jnp.float32).at[pos].set(w_all)

    total = pad_end[-1]
    nactive = (total // TM).astype(jnp.int32)
    tiles = jnp.arange(NT, dtype=jnp.int32) * TM
    raw = jnp.minimum(
        jnp.searchsorted(pad_end, tiles, side="right").astype(jnp.int32),
        NE - 1)
    last = raw[jnp.maximum(nactive - 1, 0)]
    tile_expert = jnp.where(tiles < total, raw, last)
    return (pos[:NTOK], pos[NTOK:], row_token, row_w, tile_expert,
            nactive.reshape(1))


def kernel(x, Wr, W1, b1, W2, b2):
    Bx, L, Dx = x.shape
    x2 = x.reshape(L, Dx)
    i1, i2, w1, w2 = _router(x2, Wr)
    i1, i2 = i1[:, 0], i2[:, 0]
    w1, w2 = w1[:, 0], w2[:, 0]
    posA, posB, row_token, row_w, tile_expert, nactive = _dispatch_plan(
        i1, i2, w1, w2)
    xs = _gather(x2, row_token)
    rw3 = row_w.reshape(NT, 1, TM)
    ys = _ffn(tile_expert, nactive, xs, W1, b1, W2, b2, rw3)
    out = _combine(ys, posA, posB)
    return out.reshape(Bx, L, Dx)


# speedup vs baseline: 2.5557x; 1.0049x over previous
import functools

import jax
import jax.numpy as jnp
from jax import lax
from jax.experimental import pallas as pl
from jax.experimental.pallas import tpu as pltpu
from jax.experimental.pallas import tpu_sc as plsc

D = 1024
NE = 64
NTOK = 2048
TM = 64
NPAD = 8192
NT = NPAD // TM
NW = 32
GCH = 32
CCH = 32


def _router_body(x_ref, wr_ref, i1_ref, i2_ref, w1_ref, w2_ref):
    xb = x_ref[...]
    wr = wr_ref[...]
    logits = lax.dot_general(xb, wr, (((1,), (1,)), ((), ())),
                             preferred_element_type=jnp.float32)
    iota = lax.broadcasted_iota(jnp.int32, logits.shape, 1)
    m1 = jnp.max(logits, axis=1, keepdims=True)
    i1 = jnp.min(jnp.where(logits == m1, iota, NE), axis=1, keepdims=True)
    masked = jnp.where(iota == i1, -jnp.inf, logits)
    m2 = jnp.max(masked, axis=1, keepdims=True)
    i2 = jnp.min(jnp.where(masked == m2, iota, NE), axis=1, keepdims=True)
    w1 = 1.0 / (1.0 + jnp.exp(m2 - m1))
    i1_ref[...] = i1
    i2_ref[...] = i2
    w1_ref[...] = w1
    w2_ref[...] = 1.0 - w1


def _router(x2, Wr):
    return pl.pallas_call(
        _router_body,
        out_shape=[
            jax.ShapeDtypeStruct((NTOK, 1), jnp.int32),
            jax.ShapeDtypeStruct((NTOK, 1), jnp.int32),
            jax.ShapeDtypeStruct((NTOK, 1), jnp.float32),
            jax.ShapeDtypeStruct((NTOK, 1), jnp.float32),
        ],
    )(x2, Wr)


GNB = 3
GROWS = NPAD // NW
GNCH = GROWS // GCH


def _gather_body(x_hbm, rt_hbm, out_hbm, idx_v, b0, b1, b2, g0, g1, g2,
                 s0, s1, s2):
    wid = lax.axis_index("s") * 2 + lax.axis_index("c")
    base = wid * GROWS
    bufs = (b0, b1, b2)
    gsem = (g0, g1, g2)
    wsem = (s0, s1, s2)
    pltpu.sync_copy(rt_hbm.at[pl.ds(base, GROWS)], idx_v)

    gd, wd = {}, {}

    def start_gather(c):
        b = c % GNB
        gd[c] = pltpu.async_copy(
            x_hbm.at[idx_v.at[pl.ds(c * GCH, GCH)]], bufs[b], gsem[b])

    def start_write(c):
        b = c % GNB
        wd[c] = pltpu.async_copy(
            bufs[b], out_hbm.at[pl.ds(base + c * GCH, GCH)], wsem[b])

    for c in range(GNCH):
        if c >= GNB:
            wd[c - GNB].wait()
        start_gather(c)
        if c >= 1:
            gd[c - 1].wait()
            start_write(c - 1)
    gd[GNCH - 1].wait()
    start_write(GNCH - 1)
    for c in range(max(GNCH - GNB, 0), GNCH):
        wd[c].wait()


def _gather(x2, row_token):
    f = functools.partial(
        pl.kernel,
        mesh=plsc.VectorSubcoreMesh(core_axis_name="c", subcore_axis_name="s"),
        out_type=jax.ShapeDtypeStruct((NPAD, D), jnp.float32),
        scratch_types=[
            pltpu.VMEM((GROWS,), jnp.int32),
            pltpu.VMEM((GCH, D), jnp.float32),
            pltpu.VMEM((GCH, D), jnp.float32),
            pltpu.VMEM((GCH, D), jnp.float32),
            pltpu.SemaphoreType.DMA,
            pltpu.SemaphoreType.DMA,
            pltpu.SemaphoreType.DMA,
            pltpu.SemaphoreType.DMA,
            pltpu.SemaphoreType.DMA,
            pltpu.SemaphoreType.DMA,
        ],
    )(_gather_body)
    return f(x2, row_token)


_RSQRT2 = 0.7071067811865476


def _ffn_body(te_ref, nt_ref, xs_ref, w1_ref, b1_ref, w2_ref, b2_ref, rw_ref,
              ys_ref):
    j = pl.program_id(0)

    @pl.when(j < nt_ref[0])
    def _():
        xb = xs_ref[...]
        h = lax.dot_general(xb, w1_ref[0], (((1,), (1,)), ((), ())),
                            preferred_element_type=jnp.float32)
        h = h + b1_ref[0]
        h = 0.5 * h * (1.0 + lax.erf(h * _RSQRT2))
        y = lax.dot_general(h, w2_ref[0], (((1,), (1,)), ((), ())),
                            preferred_element_type=jnp.float32)
        y = y + b2_ref[0]
        ys_ref[...] = y * rw_ref[0, 0, :][:, None]


def _ffn(tile_expert, nactive, xs, W1, b1, W2, b2, rw3):
    def _jm(j, te, nt):
        return jnp.minimum(j, nt[0] - 1)

    grid_spec = pltpu.PrefetchScalarGridSpec(
        num_scalar_prefetch=2,
        grid=(NT,),
        in_specs=[
            pl.BlockSpec((TM, D), lambda j, te, nt: (_jm(j, te, nt), 0)),
            pl.BlockSpec((1, D, D), lambda j, te, nt: (te[_jm(j, te, nt)], 0, 0)),
            pl.BlockSpec((1, 1, D), lambda j, te, nt: (te[_jm(j, te, nt)], 0, 0)),
            pl.BlockSpec((1, D, D), lambda j, te, nt: (te[_jm(j, te, nt)], 0, 0)),
            pl.BlockSpec((1, 1, D), lambda j, te, nt: (te[_jm(j, te, nt)], 0, 0)),
            pl.BlockSpec((1, 1, TM), lambda j, te, nt: (_jm(j, te, nt), 0, 0)),
        ],
        out_specs=pl.BlockSpec((TM, D), lambda j, te, nt: (_jm(j, te, nt), 0)),
    )
    return pl.pallas_call(
        _ffn_body,
        grid_spec=grid_spec,
        out_shape=jax.ShapeDtypeStruct((NPAD, D), jnp.float32),
        compiler_params=pltpu.CompilerParams(
            dimension_semantics=("arbitrary",)),
    )(tile_expert, nactive, xs, W1, b1.reshape(NE, 1, D), W2,
      b2.reshape(NE, 1, D), rw3)


def _combine_body(ys_hbm, pa_hbm, pb_hbm, out_hbm, ia_v, ib_v, ba_v, bb_v,
                  sa, sb):
    wid = lax.axis_index("s") * 2 + lax.axis_index("c")
    base = wid * (NTOK // NW)

    def chunk(c, carry):
        off = base + c * CCH
        pltpu.sync_copy(pa_hbm.at[pl.ds(off, CCH)], ia_v)
        pltpu.sync_copy(pb_hbm.at[pl.ds(off, CCH)], ib_v)
        cpa = pltpu.async_copy(ys_hbm.at[ia_v], ba_v, sa)
        cpb = pltpu.async_copy(ys_hbm.at[ib_v], bb_v, sb)
        cpa.wait()
        cpb.wait()

        def row(r, carry2):
            for i in range(D // 16):
                sl = pl.ds(i * 16, 16)
                ba_v[r, sl] = ba_v[r, sl] + bb_v[r, sl]
            return carry2

        lax.fori_loop(0, CCH, row, 0)
        pltpu.sync_copy(ba_v, out_hbm.at[pl.ds(off, CCH)])
        return carry

    lax.fori_loop(0, (NTOK // NW) // CCH, chunk, 0)


def _combine(ys, posA, posB):
    f = functools.partial(
        pl.kernel,
        mesh=plsc.VectorSubcoreMesh(core_axis_name="c", subcore_axis_name="s"),
        out_type=jax.ShapeDtypeStruct((NTOK, D), jnp.float32),
        scratch_types=[
            pltpu.VMEM((CCH,), jnp.int32),
            pltpu.VMEM((CCH,), jnp.int32),
            pltpu.VMEM((CCH, D), jnp.float32),
            pltpu.VMEM((CCH, D), jnp.float32),
            pltpu.SemaphoreType.DMA,
            pltpu.SemaphoreType.DMA,
        ],
    )(_combine_body)
    return f(ys, posA, posB)


def _dispatch_plan(i1, i2, w1, w2):
    e = jnp.concatenate([i1, i2])
    oh = (e[:, None] == jnp.arange(NE, dtype=jnp.int32)[None, :])
    cum = jnp.cumsum(oh.astype(jnp.int32), axis=0)
    counts = cum[-1]
    rank = jnp.take_along_axis(cum, e[:, None], axis=1)[:, 0] - 1
    padded = ((counts + TM - 1) // TM) * TM
    pad_end = jnp.cumsum(padded)
    pad_off = pad_end - padded
    pos = pad_off[e] + rank

    token_ids = jnp.tile(jnp.arange(NTOK, dtype=jnp.int32), 2)
    row_token = jnp.zeros((NPAD,), jnp.int32).at[pos].set(token_ids)
    w_all = jnp.concatenate([w1, w2])
    row_w = jnp.zeros((NPAD,), jnp.float32).at[pos].set(w_all)

    total = pad_end[-1]
    nactive = (total // TM).astype(jnp.int32)
    tiles = jnp.arange(NT, dtype=jnp.int32) * TM
    raw = jnp.minimum(
        jnp.searchsorted(pad_end, tiles, side="right").astype(jnp.int32),
        NE - 1)
    last = raw[jnp.maximum(nactive - 1, 0)]
    tile_expert = jnp.where(tiles < total, raw, last)
    return (pos[:NTOK], pos[NTOK:], row_token, row_w, tile_expert,
            nactive.reshape(1))


def kernel(x, Wr, W1, b1, W2, b2):
    Bx, L, Dx = x.shape
    x2 = x.reshape(L, Dx)
    i1, i2, w1, w2 = _router(x2, Wr)
    i1, i2 = i1[:, 0], i2[:, 0]
    w1, w2 = w1[:, 0], w2[:, 0]
    posA, posB, row_token, row_w, tile_expert, nactive = _dispatch_plan(
        i1, i2, w1, w2)
    xs = _gather(x2, row_token)
    rw3 = row_w.reshape(NT, 1, TM)
    ys = _ffn(tile_expert, nactive, xs, W1, b1, W2, b2, rw3)
    out = _combine(ys, posA, posB)
    return out.reshape(Bx, L, Dx)
